# baseline (speedup 1.0000x reference)
import functools

import jax
import jax.numpy as jnp
from jax import lax
from jax.experimental import pallas as pl
from jax.experimental.pallas import tpu as pltpu
from jax.experimental.pallas import tpu_sc as plsc

N = 10000
NP = 10240
E = 320000
F = 128
FH = 64
K = 5
NS = 16
C = 128
GB = 8
NG = 20
NCH = NG * GB
ET = NCH * C
EP = NS * ET
NT = NP // NS
BN = 640


def _full16(v):
    return jnp.full((16,), v, jnp.int32)


def _rsqrt16(d):
    i = plsc.bitcast(d, jnp.int32)
    i = jnp.int32(0x5F3759DF) - (i >> 1)
    y = plsc.bitcast(i, jnp.float32)
    for _ in range(3):
        y = y * (1.5 - 0.5 * d * y * y)
    return jnp.where(d > 0, y, 0.0)


def _sc_body(x_hbm, row_hbm, col_hbm, ew_hbm, v_out,
             bufA, bufB, deg_sh, dinv_sh,
             rc0, cc0, ewb0, rc1, cc1, ewb1,
             wbuf, dinv_l, G0, G1, zb, degv, dinvv, stab,
             esem, gsem0, gsem1, ssem0, ssem1):
    c = lax.axis_index("c")
    s = lax.axis_index("s")
    nsl = pl.ds(s * NT, NT)

    for i in range(16):
        stab[i, ...] = _full16(i)

    pltpu.sync_copy(x_hbm.at[c, nsl], bufA.at[nsl])
    for r in range(C):
        for f in range(FH // 16):
            zb[r, pl.ds(f * 16, 16)] = jnp.zeros((16,), jnp.float32)
    for g in range(NT // 16):
        degv[pl.ds(g * 16, 16)] = jnp.zeros((16,), jnp.float32)
    pltpu.sync_copy(degv, deg_sh.at[nsl])
    plsc.subcore_barrier()

    pltpu.sync_copy(row_hbm.at[s, 0], rc0)
    pltpu.sync_copy(ew_hbm.at[s, 0], ewb0)

    def _deg_group(g, rc_p, ewb_p, rc_o, ewb_o):
        @pl.when(g > 0)
        def _():
            pltpu.make_async_copy(row_hbm.at[s, g], rc_p, esem).wait()
            pltpu.make_async_copy(ew_hbm.at[s, g], ewb_p, esem).wait()

        @pl.when(g + 1 < NG)
        def _():
            pltpu.async_copy(row_hbm.at[s, g + 1], rc_o, esem)
            pltpu.async_copy(ew_hbm.at[s, g + 1], ewb_o, esem)

        for k in range(GB):
            pltpu.async_copy(ewb_p.at[pl.ds(k * C, C)],
                             deg_sh.at[rc_p.at[k]], gsem0, add=True)
        for k in range(GB):
            pltpu.make_async_copy(ewb_p.at[pl.ds(k * C, C)],
                                  deg_sh.at[rc_p.at[k]], gsem0).wait()

    @pl.loop(0, NG, step=2)
    def _deg(gg):
        _deg_group(gg, rc0, ewb0, rc1, ewb1)
        _deg_group(gg + 1, rc1, ewb1, rc0, ewb0)

    plsc.subcore_barrier()

    pltpu.sync_copy(deg_sh.at[nsl], degv)
    for g in range(NT // 16):
        sl = pl.ds(g * 16, 16)
        dinvv[sl] = _rsqrt16(degv[sl])
    pltpu.sync_copy(dinvv, dinv_sh.at[nsl])
    plsc.subcore_barrier()

    pltpu.sync_copy(dinv_sh, dinv_l)

    def _scale(Gb, rc, cc, ewb, kk):
        @pl.loop(0, C // 16)
        def _grp(grp):
            sl = pl.ds(grp * 16, 16)
            dr = plsc.load_gather(dinv_l, [rc[kk, sl]])
            dc = plsc.load_gather(dinv_l, [cc[kk, sl]])
            wbuf[...] = -(dr * ewb[pl.ds(kk * C + grp * 16, 16)] * dc)

            @pl.loop(0, 16, unroll=16)
            def _edge(i):
                ws = plsc.load_gather(wbuf, [stab[i, ...]])
                e = grp * 16 + i
                for f in range(FH // 16):
                    fsl = pl.ds(f * 16, 16)
                    Gb[e, fsl] = Gb[e, fsl] * ws

    for hop in range(1, K):
        src = bufA if hop % 2 == 1 else bufB
        dst = bufB if hop % 2 == 1 else bufA

        for kk in range(NT // C):
            pltpu.sync_copy(zb, dst.at[pl.ds(s * NT + kk * C, C)])
        plsc.subcore_barrier()

        pltpu.sync_copy(row_hbm.at[s, 0], rc0)
        pltpu.sync_copy(col_hbm.at[s, 0], cc0)
        pltpu.sync_copy(ew_hbm.at[s, 0], ewb0)
        pltpu.async_copy(src.at[rc0.at[0]], G0, gsem0)

        def _group(g, rc_p, cc_p, ewb_p, rc_o, cc_o, ewb_o):
            @pl.when(g > 0)
            def _():
                pltpu.make_async_copy(G1, dst.at[pl.ds(0, C)], ssem1).wait()

            @pl.when(g + 1 < NG)
            def _():
                pltpu.async_copy(row_hbm.at[s, g + 1], rc_o, esem)
                pltpu.async_copy(col_hbm.at[s, g + 1], cc_o, esem)
                pltpu.async_copy(ew_hbm.at[s, g + 1], ewb_o, esem)

            @pl.loop(0, GB, step=2)
            def _pair(k):
                pltpu.make_async_copy(src.at[pl.ds(0, C)], G0, gsem0).wait()

                @pl.when(k > 0)
                def _():
                    pltpu.make_async_copy(G1, dst.at[pl.ds(0, C)],
                                          ssem1).wait()

                pltpu.async_copy(src.at[rc_p.at[k + 1]], G1, gsem1)
                _scale(G0, rc_p, cc_p, ewb_p, k)
                pltpu.async_copy(G0, dst.at[cc_p.at[k]], ssem0, add=True)

                pltpu.make_async_copy(src.at[pl.ds(0, C)], G1, gsem1).wait()

                @pl.when(k < GB - 2)
                def _():
                    pltpu.make_async_copy(G0, dst.at[pl.ds(0, C)],
                                          ssem0).wait()
                    pltpu.async_copy(src.at[rc_p.at[k + 2]], G0, gsem0)

                _scale(G1, rc_p, cc_p, ewb_p, k + 1)
                pltpu.async_copy(G1, dst.at[cc_p.at[k + 1]], ssem1, add=True)

            @pl.when(g + 1 < NG)
            def _():
                pltpu.make_async_copy(row_hbm.at[s, g + 1], rc_o, esem).wait()
                pltpu.make_async_copy(col_hbm.at[s, g + 1], cc_o, esem).wait()
                pltpu.make_async_copy(ew_hbm.at[s, g + 1], ewb_o, esem).wait()
                pltpu.make_async_copy(G0, dst.at[pl.ds(0, C)], ssem0).wait()
                pltpu.async_copy(src.at[rc_o.at[0]], G0, gsem0)

        @pl.loop(0, NG, step=2)
        def _groups(g):
            _group(g, rc0, cc0, ewb0, rc1, cc1, ewb1)
            _group(g + 1, rc1, cc1, ewb1, rc0, cc0, ewb0)

        pltpu.make_async_copy(G0, dst.at[pl.ds(0, C)], ssem0).wait()
        pltpu.make_async_copy(G1, dst.at[pl.ds(0, C)], ssem1).wait()

        plsc.subcore_barrier()
        pltpu.sync_copy(dst.at[nsl], v_out.at[hop - 1, c, nsl])


_sc_kernel = functools.partial(
    pl.kernel,
    out_type=jax.ShapeDtypeStruct((K - 1, 2, NP, FH), jnp.float32),
    mesh=plsc.VectorSubcoreMesh(core_axis_name="c", subcore_axis_name="s",
                                num_cores=2, num_subcores=NS),
    compiler_params=pltpu.CompilerParams(needs_layout_passes=False,
                                         use_tc_tiling_on_sc=False),
    scratch_types=[
        pltpu.VMEM_SHARED((NP, FH), jnp.float32),
        pltpu.VMEM_SHARED((NP, FH), jnp.float32),
        pltpu.VMEM_SHARED((NP,), jnp.float32),
        pltpu.VMEM_SHARED((NP,), jnp.float32),
        pltpu.VMEM((GB, C), jnp.int32),
        pltpu.VMEM((GB, C), jnp.int32),
        pltpu.VMEM((GB * C,), jnp.float32),
        pltpu.VMEM((GB, C), jnp.int32),
        pltpu.VMEM((GB, C), jnp.int32),
        pltpu.VMEM((GB * C,), jnp.float32),
        pltpu.VMEM((16,), jnp.float32),
        pltpu.VMEM((NP,), jnp.float32),
        pltpu.VMEM((C, FH), jnp.float32),
        pltpu.VMEM((C, FH), jnp.float32),
        pltpu.VMEM((C, FH), jnp.float32),
        pltpu.VMEM((NT,), jnp.float32),
        pltpu.VMEM((NT,), jnp.float32),
        pltpu.VMEM((16, 16), jnp.int32),
        pltpu.SemaphoreType.DMA,
        pltpu.SemaphoreType.DMA,
        pltpu.SemaphoreType.DMA,
        pltpu.SemaphoreType.DMA,
        pltpu.SemaphoreType.DMA,
    ],
)(_sc_body)


def _mm_body(x_ref, v_ref, wc_ref, bias_ref, out_ref):
    acc = jnp.dot(x_ref[...], wc_ref[0], preferred_element_type=jnp.float32)
    for j in range(1, K):
        for cc in range(2):
            acc += jnp.dot(v_ref[j - 1, cc],
                           wc_ref[j, cc * FH:(cc + 1) * FH, :],
                           preferred_element_type=jnp.float32)
    out_ref[...] = acc + bias_ref[0][None, :]


def kernel(x, edge_weight, W, bias, edge_index):
    pad_e = EP - E
    row = jnp.concatenate([edge_index[0], jnp.zeros((pad_e,), jnp.int32)])
    col = jnp.concatenate([edge_index[1], jnp.zeros((pad_e,), jnp.int32)])
    ew = jnp.concatenate([edge_weight, jnp.zeros((pad_e,), jnp.float32)])
    row_r = row.reshape(NS, NG, GB, C)
    col_r = col.reshape(NS, NG, GB, C)
    ew_r = ew.reshape(NS, NG, GB * C)

    x_pad = jnp.pad(x, ((0, NP - N), (0, 0)))
    x_split = x_pad.reshape(NP, 2, FH).transpose(1, 0, 2)

    Wc = jnp.stack([
        W[0] - W[2] + W[4],
        W[1] - 3.0 * W[3],
        2.0 * W[2] - 8.0 * W[4],
        4.0 * W[3],
        8.0 * W[4],
    ])

    v = _sc_kernel(x_split, row_r, col_r, ew_r)

    out = pl.pallas_call(
        _mm_body,
        grid=(NP // BN,),
        in_specs=[
            pl.BlockSpec((BN, F), lambda i: (i, 0)),
            pl.BlockSpec((K - 1, 2, BN, FH), lambda i: (0, 0, i, 0)),
            pl.BlockSpec((K, F, F), lambda i: (0, 0, 0)),
            pl.BlockSpec((1, F), lambda i: (0, 0)),
        ],
        out_specs=pl.BlockSpec((BN, F), lambda i: (i, 0)),
        out_shape=jax.ShapeDtypeStruct((NP, F), jnp.float32),
    )(x_pad, v, Wc, bias.reshape(1, F))
    return out[:N]


# speedup vs baseline: 1.0343x; 1.0343x over previous
import functools

import jax
import jax.numpy as jnp
from jax import lax
from jax.experimental import pallas as pl
from jax.experimental.pallas import tpu as pltpu
from jax.experimental.pallas import tpu_sc as plsc

N = 10000
NP = 10240
E = 320000
F = 128
FH = 64
K = 5
NS = 16
C = 128
GB = 8
NG = 20
NCH = NG * GB
ET = NCH * C
EP = NS * ET
NT = NP // NS
BN = 640


def _full16(v):
    return jnp.full((16,), v, jnp.int32)


def _rsqrt16(d):
    i = plsc.bitcast(d, jnp.int32)
    i = jnp.int32(0x5F3759DF) - (i >> 1)
    y = plsc.bitcast(i, jnp.float32)
    for _ in range(3):
        y = y * (1.5 - 0.5 * d * y * y)
    return jnp.where(d > 0, y, 0.0)


def _sc_body(x_hbm, row_hbm, col_hbm, ew_hbm, v_out,
             bufA, bufB, deg_sh, dinv_sh,
             rc0, cc0, ewb0, rc1, cc1, ewb1,
             wbuf, dinv_l, G0, G1, zb, degv, dinvv, stab,
             esem, gsem0, gsem1, ssem0, ssem1):
    c = lax.axis_index("c")
    s = lax.axis_index("s")
    nsl = pl.ds(s * NT, NT)

    for i in range(16):
        stab[i, ...] = _full16(i)

    pltpu.sync_copy(x_hbm.at[c, nsl], bufA.at[nsl])
    for r in range(C):
        for f in range(FH // 16):
            zb[r, pl.ds(f * 16, 16)] = jnp.zeros((16,), jnp.float32)
    for g in range(NT // 16):
        degv[pl.ds(g * 16, 16)] = jnp.zeros((16,), jnp.float32)
    pltpu.sync_copy(degv, deg_sh.at[nsl])
    plsc.subcore_barrier()

    pltpu.sync_copy(row_hbm.at[s, 0], rc0)
    pltpu.sync_copy(ew_hbm.at[s, 0], ewb0)

    def _deg_group(g, rc_p, ewb_p, rc_o, ewb_o):
        @pl.when(g > 0)
        def _():
            pltpu.make_async_copy(row_hbm.at[s, g], rc_p, esem).wait()
            pltpu.make_async_copy(ew_hbm.at[s, g], ewb_p, esem).wait()

        @pl.when(g + 1 < NG)
        def _():
            pltpu.async_copy(row_hbm.at[s, g + 1], rc_o, esem)
            pltpu.async_copy(ew_hbm.at[s, g + 1], ewb_o, esem)

        for k in range(GB):
            pltpu.async_copy(ewb_p.at[pl.ds(k * C, C)],
                             deg_sh.at[rc_p.at[k]], gsem0, add=True)
        for k in range(GB):
            pltpu.make_async_copy(ewb_p.at[pl.ds(k * C, C)],
                                  deg_sh.at[rc_p.at[k]], gsem0).wait()

    @pl.loop(0, NG, step=2)
    def _deg(gg):
        _deg_group(gg, rc0, ewb0, rc1, ewb1)
        _deg_group(gg + 1, rc1, ewb1, rc0, ewb0)

    plsc.subcore_barrier()

    pltpu.sync_copy(deg_sh.at[nsl], degv)
    for g in range(NT // 16):
        sl = pl.ds(g * 16, 16)
        dinvv[sl] = _rsqrt16(degv[sl])
    pltpu.sync_copy(dinvv, dinv_sh.at[nsl])
    plsc.subcore_barrier()

    pltpu.sync_copy(dinv_sh, dinv_l)

    def _scale(Gb, rc, cc, ewb, kk):
        @pl.loop(0, C // 16)
        def _grp(grp):
            sl = pl.ds(grp * 16, 16)
            dr = plsc.load_gather(dinv_l, [rc[kk, sl]])
            dc = plsc.load_gather(dinv_l, [cc[kk, sl]])
            wbuf[...] = -(dr * ewb[pl.ds(kk * C + grp * 16, 16)] * dc)

            @pl.loop(0, 16, unroll=8)
            def _edge(i):
                ws = plsc.load_gather(wbuf, [stab[i, ...]])
                e = grp * 16 + i
                for f in range(FH // 16):
                    fsl = pl.ds(f * 16, 16)
                    Gb[e, fsl] = Gb[e, fsl] * ws

    for hop in range(1, K):
        src = bufA if hop % 2 == 1 else bufB
        dst = bufB if hop % 2 == 1 else bufA

        for kk in range(NT // C):
            pltpu.sync_copy(zb, dst.at[pl.ds(s * NT + kk * C, C)])
        plsc.subcore_barrier()

        pltpu.sync_copy(row_hbm.at[s, 0], rc0)
        pltpu.sync_copy(col_hbm.at[s, 0], cc0)
        pltpu.sync_copy(ew_hbm.at[s, 0], ewb0)
        pltpu.async_copy(src.at[rc0.at[0]], G0, gsem0)

        def _group(g, rc_p, cc_p, ewb_p, rc_o, cc_o, ewb_o):
            @pl.when(g > 0)
            def _():
                pltpu.make_async_copy(row_hbm.at[s, g], rc_p, esem).wait()
                pltpu.make_async_copy(col_hbm.at[s, g], cc_p, esem).wait()
                pltpu.make_async_copy(ew_hbm.at[s, g], ewb_p, esem).wait()

            @pl.when(g + 1 < NG)
            def _():
                pltpu.async_copy(row_hbm.at[s, g + 1], rc_o, esem)
                pltpu.async_copy(col_hbm.at[s, g + 1], cc_o, esem)
                pltpu.async_copy(ew_hbm.at[s, g + 1], ewb_o, esem)

            @pl.when(g > 0)
            def _():
                pltpu.make_async_copy(G0, dst.at[pl.ds(0, C)], ssem0).wait()
                pltpu.async_copy(src.at[rc_p.at[0]], G0, gsem0)

            @pl.loop(0, GB, step=2)
            def _pair(k):
                pltpu.make_async_copy(src.at[pl.ds(0, C)], G0, gsem0).wait()

                @pl.when(jnp.logical_or(g > 0, k > 0))
                def _():
                    pltpu.make_async_copy(G1, dst.at[pl.ds(0, C)],
                                          ssem1).wait()

                pltpu.async_copy(src.at[rc_p.at[k + 1]], G1, gsem1)
                _scale(G0, rc_p, cc_p, ewb_p, k)
                pltpu.async_copy(G0, dst.at[cc_p.at[k]], ssem0, add=True)

                pltpu.make_async_copy(src.at[pl.ds(0, C)], G1, gsem1).wait()

                @pl.when(k < GB - 2)
                def _():
                    pltpu.make_async_copy(G0, dst.at[pl.ds(0, C)],
                                          ssem0).wait()
                    pltpu.async_copy(src.at[rc_p.at[k + 2]], G0, gsem0)

                _scale(G1, rc_p, cc_p, ewb_p, k + 1)
                pltpu.async_copy(G1, dst.at[cc_p.at[k + 1]], ssem1, add=True)

        @pl.loop(0, NG, step=2)
        def _groups(g):
            _group(g, rc0, cc0, ewb0, rc1, cc1, ewb1)
            _group(g + 1, rc1, cc1, ewb1, rc0, cc0, ewb0)

        pltpu.make_async_copy(G0, dst.at[pl.ds(0, C)], ssem0).wait()
        pltpu.make_async_copy(G1, dst.at[pl.ds(0, C)], ssem1).wait()

        plsc.subcore_barrier()
        pltpu.sync_copy(dst.at[nsl], v_out.at[hop - 1, c, nsl])


_sc_kernel = functools.partial(
    pl.kernel,
    out_type=jax.ShapeDtypeStruct((K - 1, 2, NP, FH), jnp.float32),
    mesh=plsc.VectorSubcoreMesh(core_axis_name="c", subcore_axis_name="s",
                                num_cores=2, num_subcores=NS),
    compiler_params=pltpu.CompilerParams(needs_layout_passes=False,
                                         use_tc_tiling_on_sc=False),
    scratch_types=[
        pltpu.VMEM_SHARED((NP, FH), jnp.float32),
        pltpu.VMEM_SHARED((NP, FH), jnp.float32),
        pltpu.VMEM_SHARED((NP,), jnp.float32),
        pltpu.VMEM_SHARED((NP,), jnp.float32),
        pltpu.VMEM((GB, C), jnp.int32),
        pltpu.VMEM((GB, C), jnp.int32),
        pltpu.VMEM((GB * C,), jnp.float32),
        pltpu.VMEM((GB, C), jnp.int32),
        pltpu.VMEM((GB, C), jnp.int32),
        pltpu.VMEM((GB * C,), jnp.float32),
        pltpu.VMEM((16,), jnp.float32),
        pltpu.VMEM((NP,), jnp.float32),
        pltpu.VMEM((C, FH), jnp.float32),
        pltpu.VMEM((C, FH), jnp.float32),
        pltpu.VMEM((C, FH), jnp.float32),
        pltpu.VMEM((NT,), jnp.float32),
        pltpu.VMEM((NT,), jnp.float32),
        pltpu.VMEM((16, 16), jnp.int32),
        pltpu.SemaphoreType.DMA,
        pltpu.SemaphoreType.DMA,
        pltpu.SemaphoreType.DMA,
        pltpu.SemaphoreType.DMA,
        pltpu.SemaphoreType.DMA,
    ],
)(_sc_body)


def _mm_body(x_ref, v_ref, wc_ref, bias_ref, out_ref):
    acc = jnp.dot(x_ref[...], wc_ref[0], preferred_element_type=jnp.float32)
    for j in range(1, K):
        for cc in range(2):
            acc += jnp.dot(v_ref[j - 1, cc],
                           wc_ref[j, cc * FH:(cc + 1) * FH, :],
                           preferred_element_type=jnp.float32)
    out_ref[...] = acc + bias_ref[0][None, :]


def kernel(x, edge_weight, W, bias, edge_index):
    pad_e = EP - E
    row = jnp.concatenate([edge_index[0], jnp.zeros((pad_e,), jnp.int32)])
    col = jnp.concatenate([edge_index[1], jnp.zeros((pad_e,), jnp.int32)])
    ew = jnp.concatenate([edge_weight, jnp.zeros((pad_e,), jnp.float32)])
    row_r = row.reshape(NS, NG, GB, C)
    col_r = col.reshape(NS, NG, GB, C)
    ew_r = ew.reshape(NS, NG, GB * C)

    x_pad = jnp.pad(x, ((0, NP - N), (0, 0)))
    x_split = x_pad.reshape(NP, 2, FH).transpose(1, 0, 2)

    Wc = jnp.stack([
        W[0] - W[2] + W[4],
        W[1] - 3.0 * W[3],
        2.0 * W[2] - 8.0 * W[4],
        4.0 * W[3],
        8.0 * W[4],
    ])

    v = _sc_kernel(x_split, row_r, col_r, ew_r)

    out = pl.pallas_call(
        _mm_body,
        grid=(NP // BN,),
        in_specs=[
            pl.BlockSpec((BN, F), lambda i: (i, 0)),
            pl.BlockSpec((K - 1, 2, BN, FH), lambda i: (0, 0, i, 0)),
            pl.BlockSpec((K, F, F), lambda i: (0, 0, 0)),
            pl.BlockSpec((1, F), lambda i: (0, 0)),
        ],
        out_specs=pl.BlockSpec((BN, F), lambda i: (i, 0)),
        out_shape=jax.ShapeDtypeStruct((NP, F), jnp.float32),
    )(x_pad, v, Wc, bias.reshape(1, F))
    return out[:N]
